# baseline (speedup 1.0000x reference)
import functools

import jax
import jax.numpy as jnp
from jax import lax
from jax.experimental import pallas as pl
from jax.experimental.pallas import tpu as pltpu
from jax.experimental.pallas import tpu_sc as plsc

N_USERS = 100000
D_BIAS = 64
BATCH = 4096

_INFO = plsc.get_sparse_core_info()
_NC = _INFO.num_cores
_NS = _INFO.num_subcores
_NW = _NC * _NS
_B_PER_W = BATCH // _NW


@functools.partial(
    pl.kernel,
    mesh=plsc.VectorSubcoreMesh(core_axis_name="c", subcore_axis_name="s"),
    out_type=jax.ShapeDtypeStruct((BATCH, D_BIAS), jnp.float32),
    scratch_types=[
        pltpu.VMEM((_B_PER_W,), jnp.int32),
        pltpu.VMEM((_B_PER_W, D_BIAS), jnp.float32),
        pltpu.SemaphoreType.DMA,
    ],
    compiler_params=pltpu.CompilerParams(use_tc_tiling_on_sc=False),
)
def _sc_gather(table_hbm, idx_hbm, out_hbm, idx_v, rows_v, sem):
    wid = lax.axis_index("s") * _NC + lax.axis_index("c")
    base = wid * _B_PER_W
    pltpu.sync_copy(idx_hbm.at[pl.ds(base, _B_PER_W)], idx_v)
    pltpu.async_copy(table_hbm.at[idx_v], rows_v, sem).wait()
    pltpu.sync_copy(rows_v, out_hbm.at[pl.ds(base, _B_PER_W)])


def kernel(user_id, weight):
    return _sc_gather(weight, user_id.astype(jnp.int32))


# speedup vs baseline: 1.4245x; 1.4245x over previous
import functools

import jax
import jax.numpy as jnp
from jax import lax
from jax.experimental import pallas as pl
from jax.experimental.pallas import tpu as pltpu
from jax.experimental.pallas import tpu_sc as plsc

N_USERS = 100000
D_BIAS = 64
BATCH = 4096

_INFO = plsc.get_sparse_core_info()
_NC = _INFO.num_cores
_NS = _INFO.num_subcores
_NL = _INFO.num_lanes
_NW = _NC * _NS
_B_PER_W = BATCH // _NW


@functools.partial(
    pl.kernel,
    mesh=plsc.VectorSubcoreMesh(core_axis_name="c", subcore_axis_name="s"),
    out_type=jax.ShapeDtypeStruct((BATCH, D_BIAS), jnp.float32),
    scratch_types=[
        pltpu.VMEM((_B_PER_W,), jnp.int32),
        pltpu.VMEM((_B_PER_W, D_BIAS), jnp.float32),
        pltpu.SemaphoreType.DMA,
        pltpu.SemaphoreType.DMA,
    ],
    compiler_params=pltpu.CompilerParams(needs_layout_passes=False),
)
def _sc_gather(table_hbm, idx_hbm, out_hbm, idx_v, rows_v, sem_i, sem_r):
    wid = lax.axis_index("s") * _NC + lax.axis_index("c")
    base = wid * _B_PER_W
    cp_idx = pltpu.make_async_copy(
        idx_hbm.at[pl.ds(base, _B_PER_W)], idx_v, sem_i
    )
    cp_idx.start()
    cp_idx.wait()

    lane_iota = lax.broadcasted_iota(jnp.int32, (_NL,), 0)
    for g in range(_B_PER_W // _NL):
        chunk = idx_v[pl.ds(g * _NL, _NL)]
        for lane in range(_NL):
            d = lane_iota - lane
            onehot = 1 - jnp.minimum(d * d, 1)
            u = jnp.sum(chunk * onehot)
            pltpu.make_async_copy(
                table_hbm.at[pl.ds(u, 1)],
                rows_v.at[pl.ds(g * _NL + lane, 1)],
                sem_r,
            ).start()

    pltpu.make_async_copy(
        table_hbm.at[pl.ds(0, _B_PER_W)], rows_v, sem_r
    ).wait()

    pltpu.sync_copy(rows_v, out_hbm.at[pl.ds(base, _B_PER_W)])


def kernel(user_id, weight):
    return _sc_gather(weight, user_id.astype(jnp.int32))


# speedup vs baseline: 1.4317x; 1.0051x over previous
import functools

import jax
import jax.numpy as jnp
from jax import lax
from jax.experimental import pallas as pl
from jax.experimental.pallas import tpu as pltpu
from jax.experimental.pallas import tpu_sc as plsc

N_USERS = 100000
D_BIAS = 64
BATCH = 4096

_INFO = plsc.get_sparse_core_info()
_NC = 1
_NS = _INFO.num_subcores
_NL = _INFO.num_lanes
_NW = _NC * _NS
_B_PER_W = BATCH // _NW


@functools.partial(
    pl.kernel,
    mesh=plsc.VectorSubcoreMesh(
        core_axis_name="c", subcore_axis_name="s", num_cores=_NC
    ),
    out_type=jax.ShapeDtypeStruct((BATCH, D_BIAS), jnp.float32),
    scratch_types=[
        pltpu.VMEM((_B_PER_W,), jnp.int32),
        pltpu.VMEM((_B_PER_W, D_BIAS), jnp.float32),
        pltpu.SemaphoreType.DMA,
        pltpu.SemaphoreType.DMA,
    ],
    compiler_params=pltpu.CompilerParams(needs_layout_passes=False),
)
def _sc_gather(table_hbm, idx_hbm, out_hbm, idx_v, rows_v, sem_i, sem_r):
    wid = lax.axis_index("s") * _NC + lax.axis_index("c")
    base = wid * _B_PER_W
    cp_idx = pltpu.make_async_copy(
        idx_hbm.at[pl.ds(base, _B_PER_W)], idx_v, sem_i
    )
    cp_idx.start()
    cp_idx.wait()

    lane_iota = lax.broadcasted_iota(jnp.int32, (_NL,), 0)
    for g in range(_B_PER_W // _NL):
        chunk = idx_v[pl.ds(g * _NL, _NL)]
        for lane in range(_NL):
            d = lane_iota - lane
            onehot = 1 - jnp.minimum(d * d, 1)
            u = jnp.sum(chunk * onehot)
            pltpu.make_async_copy(
                table_hbm.at[pl.ds(u, 1)],
                rows_v.at[pl.ds(g * _NL + lane, 1)],
                sem_r,
            ).start()

    pltpu.make_async_copy(
        table_hbm.at[pl.ds(0, _B_PER_W)], rows_v, sem_r
    ).wait()

    pltpu.sync_copy(rows_v, out_hbm.at[pl.ds(base, _B_PER_W)])


def kernel(user_id, weight):
    return _sc_gather(weight, user_id.astype(jnp.int32))
